# baseline (speedup 1.0000x reference)
import functools

import jax
import jax.numpy as jnp
from jax import lax
from jax.experimental import pallas as pl
from jax.experimental.pallas import tpu as pltpu
from jax.experimental.pallas import tpu_sc as plsc

H = 768
F = 3072
E = 8
K = 2
S = 2048
C = S
T = 256
NJ = C // T
NC, NS = 2, 16
NW = NC * NS
PP = (S * K) // NW
TP = S // NW


@functools.lru_cache(maxsize=None)
def _sc_kernels():
    mesh = plsc.VectorSubcoreMesh(
        core_axis_name="c", subcore_axis_name="s", num_cores=NC, num_subcores=NS)

    @functools.partial(
        pl.kernel,
        out_type=(jax.ShapeDtypeStruct((E * C, H), jnp.float32),
                  jax.ShapeDtypeStruct((E * C, 128), jnp.float32)),
        mesh=mesh,
        scratch_types=[
            pltpu.VMEM((PP,), jnp.int32),
            pltpu.VMEM((PP,), jnp.int32),
            pltpu.VMEM((PP, H), jnp.float32),
            pltpu.VMEM((PP, 128), jnp.float32),
            pltpu.SemaphoreType.DMA,
        ],
    )
    def _sc_dispatch(x_hbm, tok_hbm, slot_hbm, gwk_hbm, xg_hbm, gws_hbm,
                     tok_v, slot_v, rows_v, gw_v, sem):
        wid = lax.axis_index("s") * NC + lax.axis_index("c")
        base = wid * PP
        pltpu.sync_copy(tok_hbm.at[pl.ds(base, PP)], tok_v)
        pltpu.sync_copy(slot_hbm.at[pl.ds(base, PP)], slot_v)
        pltpu.sync_copy(gwk_hbm.at[pl.ds(base, PP)], gw_v)
        pltpu.async_copy(x_hbm.at[tok_v], rows_v, sem).wait()
        pltpu.async_copy(rows_v, xg_hbm.at[slot_v], sem).wait()
        pltpu.async_copy(gw_v, gws_hbm.at[slot_v], sem).wait()

    @functools.partial(
        pl.kernel,
        out_type=jax.ShapeDtypeStruct((S, H), jnp.float32),
        mesh=mesh,
        scratch_types=[
            pltpu.VMEM((TP,), jnp.int32),
            pltpu.VMEM((TP,), jnp.int32),
            pltpu.VMEM((TP, H), jnp.float32),
            pltpu.SemaphoreType.DMA,
        ],
    )
    def _sc_combine(y_hbm, r1_hbm, r2_hbm, out_hbm, i1_v, i2_v, a_v, sem):
        wid = lax.axis_index("s") * NC + lax.axis_index("c")
        base = wid * TP
        pltpu.sync_copy(r1_hbm.at[pl.ds(base, TP)], i1_v)
        pltpu.sync_copy(r2_hbm.at[pl.ds(base, TP)], i2_v)
        pltpu.async_copy(y_hbm.at[i1_v], a_v, sem).wait()
        pltpu.async_copy(y_hbm.at[i2_v], a_v, sem, add=True).wait()
        pltpu.sync_copy(a_v, out_hbm.at[pl.ds(base, TP)])

    return _sc_dispatch, _sc_combine


NT = (S * K) // T + E


def _ffn_body(te_ref, tj_ref, tf_ref, ta_ref,
              xg_ref, w1_hbm, b1_ref, w2_hbm, b2_ref, gws_ref, y_ref,
              w1_buf, w2_buf, sems):
    i = pl.program_id(0)
    e = te_ref[i]

    @pl.when(i == 0)
    def _():
        pltpu.make_async_copy(w1_hbm.at[0], w1_buf.at[0], sems.at[0, 0]).start()
        pltpu.make_async_copy(w2_hbm.at[0], w2_buf.at[0], sems.at[0, 1]).start()

    @pl.when((tf_ref[i] == 1) & (e + 1 < E))
    def _():
        nb = (e + 1) % 2
        pltpu.make_async_copy(w1_hbm.at[e + 1], w1_buf.at[nb],
                              sems.at[nb, 0]).start()
        pltpu.make_async_copy(w2_hbm.at[e + 1], w2_buf.at[nb],
                              sems.at[nb, 1]).start()

    @pl.when(tf_ref[i] == 1)
    def _():
        b = e % 2
        pltpu.make_async_copy(w1_hbm.at[e], w1_buf.at[b], sems.at[b, 0]).wait()

    @pl.when(ta_ref[i] == 1)
    def _():
        b = e % 2
        xb = xg_ref[...].astype(jnp.bfloat16)
        w1 = w1_buf[b].astype(jnp.bfloat16)
        h = jnp.dot(xb, w1, preferred_element_type=jnp.float32)
        h = h + b1_ref[0]
        h = 0.5 * h * (1.0 + lax.erf(h * 0.7071067811865476))

        @pl.when(tf_ref[i] == 1)
        def _():
            pltpu.make_async_copy(w2_hbm.at[e], w2_buf.at[b], sems.at[b, 1]).wait()

        w2 = w2_buf[b].astype(jnp.bfloat16)
        y = jnp.dot(h.astype(jnp.bfloat16), w2, preferred_element_type=jnp.float32)
        y = y + b2_ref[0]
        y_ref[...] = y * gws_ref[:, 0:1]

    @pl.when((tf_ref[i] == 1) & (ta_ref[i] == 0))
    def _():
        b = e % 2
        pltpu.make_async_copy(w2_hbm.at[e], w2_buf.at[b], sems.at[b, 1]).wait()


def _tile_idx(i, te, tj, tf, ta):
    return (te[i] * NJ + tj[i], 0)


_ffn = pl.pallas_call(
    _ffn_body,
    grid_spec=pltpu.PrefetchScalarGridSpec(
        num_scalar_prefetch=4,
        grid=(NT,),
        in_specs=[
            pl.BlockSpec((T, H), _tile_idx),
            pl.BlockSpec(memory_space=pl.ANY),
            pl.BlockSpec((1, 1, F), lambda i, te, tj, tf, ta: (te[i], 0, 0)),
            pl.BlockSpec(memory_space=pl.ANY),
            pl.BlockSpec((1, 1, H), lambda i, te, tj, tf, ta: (te[i], 0, 0)),
            pl.BlockSpec((T, 128), _tile_idx),
        ],
        out_specs=pl.BlockSpec((T, H), _tile_idx),
        scratch_shapes=[
            pltpu.VMEM((2, H, F), jnp.float32),
            pltpu.VMEM((2, F, H), jnp.float32),
            pltpu.SemaphoreType.DMA((2, 2)),
        ],
    ),
    out_shape=jax.ShapeDtypeStruct((E * C, H), jnp.float32),
)


def kernel(x, Wg, bg, W1, b1, W2, b2):
    x2 = x.reshape(S, H)

    gate_logits = jnp.einsum('sh,he->se', x2, Wg) + bg
    iota_e = jnp.arange(E, dtype=jnp.int32)[None, :]
    i1 = jnp.argmax(gate_logits, axis=1).astype(jnp.int32)
    oh1 = iota_e == i1[:, None]
    i2 = jnp.argmax(jnp.where(oh1, -jnp.inf, gate_logits), axis=1).astype(jnp.int32)
    oh2 = iota_e == i2[:, None]
    keep = oh1 | oh2
    masked = jnp.where(keep, gate_logits, -jnp.inf)
    masked = masked - jnp.max(masked, axis=-1, keepdims=True)
    gw = jax.nn.softmax(masked, axis=-1)

    ohm = keep.astype(jnp.int32)
    pos_all = jnp.cumsum(ohm, axis=0) - ohm
    cnt = jnp.sum(ohm, axis=0).astype(jnp.int32)
    pos1 = jnp.sum(jnp.where(oh1, pos_all, 0), axis=1)
    pos2 = jnp.sum(jnp.where(oh2, pos_all, 0), axis=1)
    r0 = (i1 * C + pos1).astype(jnp.int32)
    r1 = (i2 * C + pos2).astype(jnp.int32)
    ar = jnp.arange(S, dtype=jnp.int32)
    tok_flat = jnp.concatenate([ar, ar])
    slot_flat = jnp.concatenate([r0, r1])
    gwk_flat = jnp.concatenate([jnp.sum(jnp.where(oh1, gw, 0.0), axis=1),
                                jnp.sum(jnp.where(oh2, gw, 0.0), axis=1)])

    nt = jnp.maximum((cnt + (T - 1)) // T, 1)
    ends = jnp.cumsum(nt)
    starts = ends - nt
    total = ends[E - 1]
    ii = jnp.arange(NT, dtype=jnp.int32)
    te = jnp.minimum(jnp.sum((ii[:, None] >= ends[None, :]).astype(jnp.int32),
                             axis=1), E - 1).astype(jnp.int32)
    tj = jnp.minimum(ii - starts[te], nt[te] - 1).astype(jnp.int32)
    tf = ((ii - starts[te]) == 0).astype(jnp.int32)
    ta = ((ii < total) & (tj * T < cnt[te])).astype(jnp.int32)

    _sc_dispatch, _sc_combine = _sc_kernels()
    gwk_b = jnp.broadcast_to(gwk_flat[:, None], (S * K, 128))
    xg, gws = _sc_dispatch(x2, tok_flat, slot_flat, gwk_b)

    y = _ffn(te, tj, tf, ta, xg, W1, b1.reshape(E, 1, F), W2,
             b2.reshape(E, 1, H), gws)

    out = _sc_combine(y, r0, r1)

    return out.reshape(1, S, H), gw.reshape(1, S, E)


# speedup vs baseline: 1.0439x; 1.0439x over previous
import functools

import jax
import jax.numpy as jnp
from jax import lax
from jax.experimental import pallas as pl
from jax.experimental.pallas import tpu as pltpu
from jax.experimental.pallas import tpu_sc as plsc

H = 768
F = 3072
E = 8
K = 2
S = 2048
C = S
T = 256
NJ = C // T
NC, NS = 2, 16
NW = NC * NS
PP = (S * K) // NW
TP = S // NW


@functools.lru_cache(maxsize=None)
def _sc_kernels():
    mesh = plsc.VectorSubcoreMesh(
        core_axis_name="c", subcore_axis_name="s", num_cores=NC, num_subcores=NS)

    @functools.partial(
        pl.kernel,
        out_type=(jax.ShapeDtypeStruct((E * C, H), jnp.float32),
                  jax.ShapeDtypeStruct((E * C, 128), jnp.float32)),
        mesh=mesh,
        scratch_types=[
            pltpu.VMEM((PP,), jnp.int32),
            pltpu.VMEM((PP,), jnp.int32),
            pltpu.VMEM((PP, H), jnp.float32),
            pltpu.VMEM((PP, 128), jnp.float32),
            pltpu.SemaphoreType.DMA,
        ],
    )
    def _sc_dispatch(x_hbm, tok_hbm, slot_hbm, gwk_hbm, xg_hbm, gws_hbm,
                     tok_v, slot_v, rows_v, gw_v, sem):
        wid = lax.axis_index("s") * NC + lax.axis_index("c")
        base = wid * PP
        pltpu.sync_copy(tok_hbm.at[pl.ds(base, PP)], tok_v)
        pltpu.sync_copy(slot_hbm.at[pl.ds(base, PP)], slot_v)
        pltpu.sync_copy(gwk_hbm.at[pl.ds(base, PP)], gw_v)
        pltpu.async_copy(x_hbm.at[tok_v], rows_v, sem).wait()
        pltpu.async_copy(rows_v, xg_hbm.at[slot_v], sem).wait()
        pltpu.async_copy(gw_v, gws_hbm.at[slot_v], sem).wait()

    @functools.partial(
        pl.kernel,
        out_type=jax.ShapeDtypeStruct((S, H), jnp.float32),
        mesh=mesh,
        scratch_types=[
            pltpu.VMEM((TP,), jnp.int32),
            pltpu.VMEM((TP,), jnp.int32),
            pltpu.VMEM((TP, H), jnp.float32),
            pltpu.VMEM((TP, H), jnp.float32),
            pltpu.SemaphoreType.DMA,
        ],
    )
    def _sc_combine(y_hbm, r1_hbm, r2_hbm, out_hbm, i1_v, i2_v, a_v, b_v, sem):
        wid = lax.axis_index("s") * NC + lax.axis_index("c")
        base = wid * TP
        pltpu.sync_copy(r1_hbm.at[pl.ds(base, TP)], i1_v)
        pltpu.sync_copy(r2_hbm.at[pl.ds(base, TP)], i2_v)
        pltpu.async_copy(y_hbm.at[i1_v], a_v, sem).wait()
        pltpu.async_copy(y_hbm.at[i2_v], b_v, sem).wait()

        def _add_row(t, carry):
            for c0 in range(0, H, 16):
                a_v[t, pl.ds(c0, 16)] = a_v[t, pl.ds(c0, 16)] + b_v[t, pl.ds(c0, 16)]
            return carry

        lax.fori_loop(0, TP, _add_row, 0)
        pltpu.sync_copy(a_v, out_hbm.at[pl.ds(base, TP)])

    return _sc_dispatch, _sc_combine


NT = (S * K) // T + E


def _ffn_body(te_ref, tj_ref, tf_ref, ta_ref,
              xg_ref, w1_hbm, b1_ref, w2_hbm, b2_ref, gws_ref, y_ref,
              w1_buf, w2_buf, sems):
    i = pl.program_id(0)
    e = te_ref[i]

    @pl.when(i == 0)
    def _():
        pltpu.make_async_copy(w1_hbm.at[0], w1_buf.at[0], sems.at[0, 0]).start()
        pltpu.make_async_copy(w2_hbm.at[0], w2_buf.at[0], sems.at[0, 1]).start()

    @pl.when((tf_ref[i] == 1) & (e + 1 < E))
    def _():
        nb = (e + 1) % 2
        pltpu.make_async_copy(w1_hbm.at[e + 1], w1_buf.at[nb],
                              sems.at[nb, 0]).start()
        pltpu.make_async_copy(w2_hbm.at[e + 1], w2_buf.at[nb],
                              sems.at[nb, 1]).start()

    @pl.when(tf_ref[i] == 1)
    def _():
        b = e % 2
        pltpu.make_async_copy(w1_hbm.at[e], w1_buf.at[b], sems.at[b, 0]).wait()
        pltpu.make_async_copy(w2_hbm.at[e], w2_buf.at[b], sems.at[b, 1]).wait()

    @pl.when(ta_ref[i] == 1)
    def _():
        b = e % 2
        xb = xg_ref[...].astype(jnp.bfloat16)
        w1 = w1_buf[b].astype(jnp.bfloat16)
        h = jnp.dot(xb, w1, preferred_element_type=jnp.float32)
        h = h + b1_ref[0]
        h = 0.5 * h * (1.0 + lax.erf(h * 0.7071067811865476))
        w2 = w2_buf[b].astype(jnp.bfloat16)
        y = jnp.dot(h.astype(jnp.bfloat16), w2, preferred_element_type=jnp.float32)
        y = y + b2_ref[0]
        y_ref[...] = y * gws_ref[:, 0:1]


def _tile_idx(i, te, tj, tf, ta):
    return (te[i] * NJ + tj[i], 0)


_ffn = pl.pallas_call(
    _ffn_body,
    grid_spec=pltpu.PrefetchScalarGridSpec(
        num_scalar_prefetch=4,
        grid=(NT,),
        in_specs=[
            pl.BlockSpec((T, H), _tile_idx),
            pl.BlockSpec(memory_space=pl.ANY),
            pl.BlockSpec((1, 1, F), lambda i, te, tj, tf, ta: (te[i], 0, 0)),
            pl.BlockSpec(memory_space=pl.ANY),
            pl.BlockSpec((1, 1, H), lambda i, te, tj, tf, ta: (te[i], 0, 0)),
            pl.BlockSpec((T, 128), _tile_idx),
        ],
        out_specs=pl.BlockSpec((T, H), _tile_idx),
        scratch_shapes=[
            pltpu.VMEM((2, H, F), jnp.float32),
            pltpu.VMEM((2, F, H), jnp.float32),
            pltpu.SemaphoreType.DMA((2, 2)),
        ],
    ),
    out_shape=jax.ShapeDtypeStruct((E * C, H), jnp.float32),
)


def kernel(x, Wg, bg, W1, b1, W2, b2):
    x2 = x.reshape(S, H)

    gate_logits = jnp.einsum('sh,he->se', x2, Wg) + bg
    iota_e = jnp.arange(E, dtype=jnp.int32)[None, :]
    i1 = jnp.argmax(gate_logits, axis=1).astype(jnp.int32)
    oh1 = iota_e == i1[:, None]
    i2 = jnp.argmax(jnp.where(oh1, -jnp.inf, gate_logits), axis=1).astype(jnp.int32)
    oh2 = iota_e == i2[:, None]
    keep = oh1 | oh2
    masked = jnp.where(keep, gate_logits, -jnp.inf)
    masked = masked - jnp.max(masked, axis=-1, keepdims=True)
    gw = jax.nn.softmax(masked, axis=-1)

    ohm = keep.astype(jnp.int32)
    pos_all = jnp.cumsum(ohm, axis=0) - ohm
    cnt = jnp.sum(ohm, axis=0).astype(jnp.int32)
    pos1 = jnp.sum(jnp.where(oh1, pos_all, 0), axis=1)
    pos2 = jnp.sum(jnp.where(oh2, pos_all, 0), axis=1)
    r0 = (i1 * C + pos1).astype(jnp.int32)
    r1 = (i2 * C + pos2).astype(jnp.int32)
    ar = jnp.arange(S, dtype=jnp.int32)
    tok_flat = jnp.concatenate([ar, ar])
    slot_flat = jnp.concatenate([r0, r1])
    gwk_flat = jnp.concatenate([jnp.sum(jnp.where(oh1, gw, 0.0), axis=1),
                                jnp.sum(jnp.where(oh2, gw, 0.0), axis=1)])

    nt = jnp.maximum((cnt + (T - 1)) // T, 1)
    ends = jnp.cumsum(nt)
    starts = ends - nt
    total = ends[E - 1]
    ii = jnp.arange(NT, dtype=jnp.int32)
    te = jnp.minimum(jnp.sum((ii[:, None] >= ends[None, :]).astype(jnp.int32),
                             axis=1), E - 1).astype(jnp.int32)
    tj = jnp.minimum(ii - starts[te], nt[te] - 1).astype(jnp.int32)
    tf = ((ii - starts[te]) == 0).astype(jnp.int32)
    ta = ((ii < total) & (tj * T < cnt[te])).astype(jnp.int32)

    _sc_dispatch, _sc_combine = _sc_kernels()
    gwk_b = jnp.broadcast_to(gwk_flat[:, None], (S * K, 128))
    xg, gws = _sc_dispatch(x2, tok_flat, slot_flat, gwk_b)

    y = _ffn(te, tj, tf, ta, xg, W1, b1.reshape(E, 1, F), W2,
             b2.reshape(E, 1, H), gws)

    out = _sc_combine(y, r0, r1)

    return out.reshape(1, S, H), gw.reshape(1, S, E)


# speedup vs baseline: 1.0584x; 1.0139x over previous
import functools

import jax
import jax.numpy as jnp
from jax import lax
from jax.experimental import pallas as pl
from jax.experimental.pallas import tpu as pltpu
from jax.experimental.pallas import tpu_sc as plsc

H = 768
F = 3072
E = 8
K = 2
S = 2048
C = S
T = 256
NJ = C // T
NC, NS = 2, 16
NW = NC * NS
PP = (S * K) // NW
TP = S // NW


@functools.lru_cache(maxsize=None)
def _sc_kernels():
    mesh = plsc.VectorSubcoreMesh(
        core_axis_name="c", subcore_axis_name="s", num_cores=NC, num_subcores=NS)

    @functools.partial(
        pl.kernel,
        out_type=(jax.ShapeDtypeStruct((E * C, H), jnp.float32),
                  jax.ShapeDtypeStruct((E * C, 128), jnp.float32)),
        mesh=mesh,
        scratch_types=[
            pltpu.VMEM((TP,), jnp.int32),
            pltpu.VMEM((TP,), jnp.int32),
            pltpu.VMEM((TP, H), jnp.float32),
            pltpu.VMEM((TP, 128), jnp.float32),
            pltpu.VMEM((TP, 128), jnp.float32),
            pltpu.SemaphoreType.DMA,
            pltpu.SemaphoreType.DMA,
        ],
    )
    def _sc_dispatch(x_hbm, r1_hbm, r2_hbm, gw1_hbm, gw2_hbm, xg_hbm, gws_hbm,
                     s1_v, s2_v, rows_v, gw1_v, gw2_v, sem, sem2):
        wid = lax.axis_index("s") * NC + lax.axis_index("c")
        base = wid * TP
        pltpu.sync_copy(r1_hbm.at[pl.ds(base, TP)], s1_v)
        pltpu.sync_copy(r2_hbm.at[pl.ds(base, TP)], s2_v)
        pltpu.sync_copy(gw1_hbm.at[pl.ds(base, TP)], gw1_v)
        pltpu.sync_copy(gw2_hbm.at[pl.ds(base, TP)], gw2_v)
        pltpu.sync_copy(x_hbm.at[pl.ds(base, TP)], rows_v)
        c1 = pltpu.async_copy(rows_v, xg_hbm.at[s1_v], sem)
        c2 = pltpu.async_copy(rows_v, xg_hbm.at[s2_v], sem2)
        c3 = pltpu.async_copy(gw1_v, gws_hbm.at[s1_v], sem)
        c4 = pltpu.async_copy(gw2_v, gws_hbm.at[s2_v], sem2)
        c1.wait()
        c2.wait()
        c3.wait()
        c4.wait()

    @functools.partial(
        pl.kernel,
        out_type=jax.ShapeDtypeStruct((S, H), jnp.float32),
        mesh=mesh,
        scratch_types=[
            pltpu.VMEM((TP,), jnp.int32),
            pltpu.VMEM((TP,), jnp.int32),
            pltpu.VMEM((TP, H), jnp.float32),
            pltpu.VMEM((TP, H), jnp.float32),
            pltpu.SemaphoreType.DMA,
        ],
    )
    def _sc_combine(y_hbm, r1_hbm, r2_hbm, out_hbm, i1_v, i2_v, a_v, b_v, sem):
        wid = lax.axis_index("s") * NC + lax.axis_index("c")
        base = wid * TP
        pltpu.sync_copy(r1_hbm.at[pl.ds(base, TP)], i1_v)
        pltpu.sync_copy(r2_hbm.at[pl.ds(base, TP)], i2_v)
        pltpu.async_copy(y_hbm.at[i1_v], a_v, sem).wait()
        pltpu.async_copy(y_hbm.at[i2_v], b_v, sem).wait()

        def _add_row(t, carry):
            for c0 in range(0, H, 16):
                a_v[t, pl.ds(c0, 16)] = a_v[t, pl.ds(c0, 16)] + b_v[t, pl.ds(c0, 16)]
            return carry

        lax.fori_loop(0, TP, _add_row, 0)
        pltpu.sync_copy(a_v, out_hbm.at[pl.ds(base, TP)])

    return _sc_dispatch, _sc_combine


NT = (S * K) // T + E


def _ffn_body(te_ref, tj_ref, tf_ref, ta_ref,
              xg_ref, w1_hbm, b1_ref, w2_hbm, b2_ref, gws_ref, y_ref,
              w1_buf, w2_buf, sems):
    i = pl.program_id(0)
    e = te_ref[i]

    @pl.when(i == 0)
    def _():
        pltpu.make_async_copy(w1_hbm.at[0], w1_buf.at[0], sems.at[0, 0]).start()
        pltpu.make_async_copy(w2_hbm.at[0], w2_buf.at[0], sems.at[0, 1]).start()

    @pl.when((tf_ref[i] == 1) & (e + 1 < E))
    def _():
        nb = (e + 1) % 2
        pltpu.make_async_copy(w1_hbm.at[e + 1], w1_buf.at[nb],
                              sems.at[nb, 0]).start()
        pltpu.make_async_copy(w2_hbm.at[e + 1], w2_buf.at[nb],
                              sems.at[nb, 1]).start()

    @pl.when(tf_ref[i] == 1)
    def _():
        b = e % 2
        pltpu.make_async_copy(w1_hbm.at[e], w1_buf.at[b], sems.at[b, 0]).wait()
        pltpu.make_async_copy(w2_hbm.at[e], w2_buf.at[b], sems.at[b, 1]).wait()

    @pl.when(ta_ref[i] == 1)
    def _():
        b = e % 2
        xb = xg_ref[...].astype(jnp.bfloat16)
        w1 = w1_buf[b].astype(jnp.bfloat16)
        h = jnp.dot(xb, w1, preferred_element_type=jnp.float32)
        h = h + b1_ref[0]
        h = 0.5 * h * (1.0 + lax.erf(h * 0.7071067811865476))
        w2 = w2_buf[b].astype(jnp.bfloat16)
        y = jnp.dot(h.astype(jnp.bfloat16), w2, preferred_element_type=jnp.float32)
        y = y + b2_ref[0]
        y_ref[...] = y * gws_ref[:, 0:1]


def _tile_idx(i, te, tj, tf, ta):
    return (te[i] * NJ + tj[i], 0)


_ffn = pl.pallas_call(
    _ffn_body,
    grid_spec=pltpu.PrefetchScalarGridSpec(
        num_scalar_prefetch=4,
        grid=(NT,),
        in_specs=[
            pl.BlockSpec((T, H), _tile_idx),
            pl.BlockSpec(memory_space=pl.ANY),
            pl.BlockSpec((1, 1, F), lambda i, te, tj, tf, ta: (te[i], 0, 0)),
            pl.BlockSpec(memory_space=pl.ANY),
            pl.BlockSpec((1, 1, H), lambda i, te, tj, tf, ta: (te[i], 0, 0)),
            pl.BlockSpec((T, 128), _tile_idx),
        ],
        out_specs=pl.BlockSpec((T, H), _tile_idx),
        scratch_shapes=[
            pltpu.VMEM((2, H, F), jnp.float32),
            pltpu.VMEM((2, F, H), jnp.float32),
            pltpu.SemaphoreType.DMA((2, 2)),
        ],
    ),
    out_shape=jax.ShapeDtypeStruct((E * C, H), jnp.float32),
)


def kernel(x, Wg, bg, W1, b1, W2, b2):
    x2 = x.reshape(S, H)

    gate_logits = jnp.einsum('sh,he->se', x2, Wg) + bg
    iota_e = jnp.arange(E, dtype=jnp.int32)[None, :]
    i1 = jnp.argmax(gate_logits, axis=1).astype(jnp.int32)
    oh1 = iota_e == i1[:, None]
    i2 = jnp.argmax(jnp.where(oh1, -jnp.inf, gate_logits), axis=1).astype(jnp.int32)
    oh2 = iota_e == i2[:, None]
    keep = oh1 | oh2
    masked = jnp.where(keep, gate_logits, -jnp.inf)
    masked = masked - jnp.max(masked, axis=-1, keepdims=True)
    gw = jax.nn.softmax(masked, axis=-1)

    ohm = keep.astype(jnp.int32)
    pos_all = jnp.cumsum(ohm, axis=0) - ohm
    cnt = jnp.sum(ohm, axis=0).astype(jnp.int32)
    pos1 = jnp.sum(jnp.where(oh1, pos_all, 0), axis=1)
    pos2 = jnp.sum(jnp.where(oh2, pos_all, 0), axis=1)
    r0 = (i1 * C + pos1).astype(jnp.int32)
    r1 = (i2 * C + pos2).astype(jnp.int32)
    gw1 = jnp.sum(jnp.where(oh1, gw, 0.0), axis=1)
    gw2 = jnp.sum(jnp.where(oh2, gw, 0.0), axis=1)

    nt = jnp.maximum((cnt + (T - 1)) // T, 1)
    ends = jnp.cumsum(nt)
    starts = ends - nt
    total = ends[E - 1]
    ii = jnp.arange(NT, dtype=jnp.int32)
    te = jnp.minimum(jnp.sum((ii[:, None] >= ends[None, :]).astype(jnp.int32),
                             axis=1), E - 1).astype(jnp.int32)
    tj = jnp.minimum(ii - starts[te], nt[te] - 1).astype(jnp.int32)
    tf = ((ii - starts[te]) == 0).astype(jnp.int32)
    ta = ((ii < total) & (tj * T < cnt[te])).astype(jnp.int32)

    _sc_dispatch, _sc_combine = _sc_kernels()
    gw1_b = jnp.broadcast_to(gw1[:, None], (S, 128))
    gw2_b = jnp.broadcast_to(gw2[:, None], (S, 128))
    xg, gws = _sc_dispatch(x2, r0, r1, gw1_b, gw2_b)

    y = _ffn(te, tj, tf, ta, xg, W1, b1.reshape(E, 1, F), W2,
             b2.reshape(E, 1, H), gws)

    out = _sc_combine(y, r0, r1)

    return out.reshape(1, S, H), gw.reshape(1, S, E)


# speedup vs baseline: 1.0590x; 1.0006x over previous
import functools

import jax
import jax.numpy as jnp
from jax import lax
from jax.experimental import pallas as pl
from jax.experimental.pallas import tpu as pltpu
from jax.experimental.pallas import tpu_sc as plsc

H = 768
F = 3072
E = 8
K = 2
S = 2048
C = S
T = 256
NJ = C // T
NC, NS = 2, 16
NW = NC * NS
PP = (S * K) // NW
TP = S // NW


@functools.lru_cache(maxsize=None)
def _sc_kernels():
    mesh = plsc.VectorSubcoreMesh(
        core_axis_name="c", subcore_axis_name="s", num_cores=NC, num_subcores=NS)

    @functools.partial(
        pl.kernel,
        out_type=(jax.ShapeDtypeStruct((E * C, H), jnp.float32),
                  jax.ShapeDtypeStruct((E * C, 128), jnp.float32)),
        mesh=mesh,
        scratch_types=[
            pltpu.VMEM((TP,), jnp.int32),
            pltpu.VMEM((TP,), jnp.int32),
            pltpu.VMEM((TP, H), jnp.float32),
            pltpu.VMEM((TP, 128), jnp.float32),
            pltpu.VMEM((TP, 128), jnp.float32),
            pltpu.SemaphoreType.DMA,
            pltpu.SemaphoreType.DMA,
        ],
    )
    def _sc_dispatch(x_hbm, r1_hbm, r2_hbm, gw1_hbm, gw2_hbm, xg_hbm, gws_hbm,
                     s1_v, s2_v, rows_v, gw1_v, gw2_v, sem, sem2):
        wid = lax.axis_index("s") * NC + lax.axis_index("c")
        base = wid * TP
        pltpu.sync_copy(r1_hbm.at[pl.ds(base, TP)], s1_v)
        pltpu.sync_copy(r2_hbm.at[pl.ds(base, TP)], s2_v)
        pltpu.sync_copy(gw1_hbm.at[pl.ds(base, TP)], gw1_v)
        pltpu.sync_copy(gw2_hbm.at[pl.ds(base, TP)], gw2_v)
        pltpu.sync_copy(x_hbm.at[pl.ds(base, TP)], rows_v)
        c1 = pltpu.async_copy(rows_v, xg_hbm.at[s1_v], sem)
        c2 = pltpu.async_copy(rows_v, xg_hbm.at[s2_v], sem2)
        c3 = pltpu.async_copy(gw1_v, gws_hbm.at[s1_v], sem)
        c4 = pltpu.async_copy(gw2_v, gws_hbm.at[s2_v], sem2)
        c1.wait()
        c2.wait()
        c3.wait()
        c4.wait()

    @functools.partial(
        pl.kernel,
        out_type=jax.ShapeDtypeStruct((S, H), jnp.float32),
        mesh=mesh,
        scratch_types=[
            pltpu.VMEM((TP,), jnp.int32),
            pltpu.VMEM((TP,), jnp.int32),
            pltpu.VMEM((TP, H), jnp.float32),
            pltpu.VMEM((TP, H), jnp.float32),
            pltpu.SemaphoreType.DMA,
        ],
    )
    def _sc_combine(y_hbm, r1_hbm, r2_hbm, out_hbm, i1_v, i2_v, a_v, b_v, sem):
        wid = lax.axis_index("s") * NC + lax.axis_index("c")
        base = wid * TP
        pltpu.sync_copy(r1_hbm.at[pl.ds(base, TP)], i1_v)
        pltpu.sync_copy(r2_hbm.at[pl.ds(base, TP)], i2_v)
        pltpu.async_copy(y_hbm.at[i1_v], a_v, sem).wait()
        pltpu.async_copy(y_hbm.at[i2_v], b_v, sem).wait()

        def _add_row(t, carry):
            for c0 in range(0, H, 16):
                a_v[t, pl.ds(c0, 16)] = a_v[t, pl.ds(c0, 16)] + b_v[t, pl.ds(c0, 16)]
            return carry

        lax.fori_loop(0, TP, _add_row, 0)
        pltpu.sync_copy(a_v, out_hbm.at[pl.ds(base, TP)])

    return _sc_dispatch, _sc_combine


NT = (S * K) // T + E


def _ffn_body(te_ref, tj_ref, tf_ref, ta_ref,
              xg_ref, w1_hbm, b1_ref, w2_hbm, b2_ref, gws_ref, y_ref,
              w1_buf, w2_buf, sems):
    i = pl.program_id(0)
    e = te_ref[i]

    def _w_start(ee, bb):
        pltpu.make_async_copy(w1_hbm.at[ee, pl.ds(0, H // 2)],
                              w1_buf.at[bb, pl.ds(0, H // 2)],
                              sems.at[bb, 0]).start()
        pltpu.make_async_copy(w1_hbm.at[ee, pl.ds(H // 2, H // 2)],
                              w1_buf.at[bb, pl.ds(H // 2, H // 2)],
                              sems.at[bb, 1]).start()
        pltpu.make_async_copy(w2_hbm.at[ee, pl.ds(0, F // 2)],
                              w2_buf.at[bb, pl.ds(0, F // 2)],
                              sems.at[bb, 2]).start()
        pltpu.make_async_copy(w2_hbm.at[ee, pl.ds(F // 2, F // 2)],
                              w2_buf.at[bb, pl.ds(F // 2, F // 2)],
                              sems.at[bb, 3]).start()

    def _w_wait(ee, bb):
        pltpu.make_async_copy(w1_hbm.at[ee, pl.ds(0, H // 2)],
                              w1_buf.at[bb, pl.ds(0, H // 2)],
                              sems.at[bb, 0]).wait()
        pltpu.make_async_copy(w1_hbm.at[ee, pl.ds(H // 2, H // 2)],
                              w1_buf.at[bb, pl.ds(H // 2, H // 2)],
                              sems.at[bb, 1]).wait()
        pltpu.make_async_copy(w2_hbm.at[ee, pl.ds(0, F // 2)],
                              w2_buf.at[bb, pl.ds(0, F // 2)],
                              sems.at[bb, 2]).wait()
        pltpu.make_async_copy(w2_hbm.at[ee, pl.ds(F // 2, F // 2)],
                              w2_buf.at[bb, pl.ds(F // 2, F // 2)],
                              sems.at[bb, 3]).wait()

    @pl.when(i == 0)
    def _():
        _w_start(0, 0)

    @pl.when((tf_ref[i] == 1) & (e + 1 < E))
    def _():
        _w_start(e + 1, (e + 1) % 2)

    @pl.when(tf_ref[i] == 1)
    def _():
        _w_wait(e, e % 2)

    @pl.when(ta_ref[i] == 1)
    def _():
        b = e % 2
        xb = xg_ref[...].astype(jnp.bfloat16)
        w1 = w1_buf[b].astype(jnp.bfloat16)
        h = jnp.dot(xb, w1, preferred_element_type=jnp.float32)
        h = h + b1_ref[0]
        h = 0.5 * h * (1.0 + lax.erf(h * 0.7071067811865476))
        w2 = w2_buf[b].astype(jnp.bfloat16)
        y = jnp.dot(h.astype(jnp.bfloat16), w2, preferred_element_type=jnp.float32)
        y = y + b2_ref[0]
        y_ref[...] = y * gws_ref[:, 0:1]


def _tile_idx(i, te, tj, tf, ta):
    return (te[i] * NJ + tj[i], 0)


_ffn = pl.pallas_call(
    _ffn_body,
    grid_spec=pltpu.PrefetchScalarGridSpec(
        num_scalar_prefetch=4,
        grid=(NT,),
        in_specs=[
            pl.BlockSpec((T, H), _tile_idx),
            pl.BlockSpec(memory_space=pl.ANY),
            pl.BlockSpec((1, 1, F), lambda i, te, tj, tf, ta: (te[i], 0, 0)),
            pl.BlockSpec(memory_space=pl.ANY),
            pl.BlockSpec((1, 1, H), lambda i, te, tj, tf, ta: (te[i], 0, 0)),
            pl.BlockSpec((T, 128), _tile_idx),
        ],
        out_specs=pl.BlockSpec((T, H), _tile_idx),
        scratch_shapes=[
            pltpu.VMEM((2, H, F), jnp.float32),
            pltpu.VMEM((2, F, H), jnp.float32),
            pltpu.SemaphoreType.DMA((2, 4)),
        ],
    ),
    out_shape=jax.ShapeDtypeStruct((E * C, H), jnp.float32),
)


def kernel(x, Wg, bg, W1, b1, W2, b2):
    x2 = x.reshape(S, H)

    gate_logits = jnp.einsum('sh,he->se', x2, Wg) + bg
    iota_e = jnp.arange(E, dtype=jnp.int32)[None, :]
    i1 = jnp.argmax(gate_logits, axis=1).astype(jnp.int32)
    oh1 = iota_e == i1[:, None]
    i2 = jnp.argmax(jnp.where(oh1, -jnp.inf, gate_logits), axis=1).astype(jnp.int32)
    oh2 = iota_e == i2[:, None]
    keep = oh1 | oh2
    masked = jnp.where(keep, gate_logits, -jnp.inf)
    masked = masked - jnp.max(masked, axis=-1, keepdims=True)
    gw = jax.nn.softmax(masked, axis=-1)

    ohm = keep.astype(jnp.int32)
    pos_all = jnp.cumsum(ohm, axis=0) - ohm
    cnt = jnp.sum(ohm, axis=0).astype(jnp.int32)
    pos1 = jnp.sum(jnp.where(oh1, pos_all, 0), axis=1)
    pos2 = jnp.sum(jnp.where(oh2, pos_all, 0), axis=1)
    r0 = (i1 * C + pos1).astype(jnp.int32)
    r1 = (i2 * C + pos2).astype(jnp.int32)
    gw1 = jnp.sum(jnp.where(oh1, gw, 0.0), axis=1)
    gw2 = jnp.sum(jnp.where(oh2, gw, 0.0), axis=1)

    nt = jnp.maximum((cnt + (T - 1)) // T, 1)
    ends = jnp.cumsum(nt)
    starts = ends - nt
    total = ends[E - 1]
    ii = jnp.arange(NT, dtype=jnp.int32)
    te = jnp.minimum(jnp.sum((ii[:, None] >= ends[None, :]).astype(jnp.int32),
                             axis=1), E - 1).astype(jnp.int32)
    tj = jnp.minimum(ii - starts[te], nt[te] - 1).astype(jnp.int32)
    tf = ((ii - starts[te]) == 0).astype(jnp.int32)
    ta = ((ii < total) & (tj * T < cnt[te])).astype(jnp.int32)

    _sc_dispatch, _sc_combine = _sc_kernels()
    gw1_b = jnp.broadcast_to(gw1[:, None], (S, 128))
    gw2_b = jnp.broadcast_to(gw2[:, None], (S, 128))
    xg, gws = _sc_dispatch(x2, r0, r1, gw1_b, gw2_b)

    y = _ffn(te, tj, tf, ta, xg, W1, b1.reshape(E, 1, F), W2,
             b2.reshape(E, 1, H), gws)

    out = _sc_combine(y, r0, r1)

    return out.reshape(1, S, H), gw.reshape(1, S, E)
